# baseline (speedup 1.0000x reference)
import functools

import jax
import jax.numpy as jnp
from jax import lax
from jax.experimental import pallas as pl
from jax.experimental.pallas import tpu as pltpu
from jax.experimental.pallas import tpu_sc as plsc

D_MODEL = 64
D_PAD = 128
SCALE = 8.0

_info = plsc.get_sparse_core_info()
_NC, _NS, _L = _info.num_cores, _info.num_subcores, _info.num_lanes
_NW = _NC * _NS

CHUNK = 128
BV = 4096


def _prep_body(tt_ref, out_ref):
    at = jnp.transpose(tt_ref[...]) * SCALE
    out_ref[...] = jnp.concatenate([at, at], axis=1)


def _prep_table(table):
    v = table.shape[0]
    tt = table.T
    return pl.pallas_call(
        _prep_body,
        grid=(pl.cdiv(v, BV),),
        in_specs=[pl.BlockSpec((D_MODEL, BV), lambda i: (0, i))],
        out_specs=pl.BlockSpec((BV, D_PAD), lambda i: (i, 0)),
        out_shape=jax.ShapeDtypeStruct((v, D_PAD), jnp.float32),
    )(tt)


def _gather_body(idx_hbm, table_hbm, out_hbm, idx_v, buf0, buf1, outb,
                 sem0, sem1, *, b_per_w, n_chunks, n_bblk):
    wid = lax.axis_index("s") * _NC + lax.axis_index("c")
    base = wid * b_per_w
    k0 = wid * n_chunks
    pltpu.sync_copy(idx_hbm.at[pl.ds(base, b_per_w)], idx_v)

    bufs = (buf0, buf1)
    sems = (sem0, sem1)
    iota = lax.iota(jnp.int32, _L)

    def start_gather(g, b):
        pltpu.make_async_copy(
            table_hbm.at[idx_v.at[pl.ds(g * CHUNK, CHUNK)]], bufs[b], sems[b]
        ).start()

    def finish(g, b):
        pltpu.make_async_copy(
            table_hbm.at[idx_v.at[pl.ds(g * CHUNK, CHUNK)]], bufs[b], sems[b]
        ).wait()
        buf = bufs[b]

        def xpose(g16, carry):
            rowvec = iota + g16 * _L
            for f in range(D_MODEL):
                v = plsc.load_gather(buf, [rowvec, iota * 0 + f])
                outb[f // 8, f % 8, pl.ds(g16 * _L, _L)] = v
            return carry

        lax.fori_loop(0, CHUNK // _L, xpose, 0)

        k = k0 + g
        j = k // n_bblk
        bblk = k % n_bblk
        pltpu.sync_copy(outb, out_hbm.at[j, :, bblk])
        @pl.when(g + 2 < n_chunks)
        def _():
            start_gather(g + 2, b)

    start_gather(0, 0)
    start_gather(1, 1)

    def body(p, carry):
        g = p * 2
        finish(g, 0)
        finish(g + 1, 1)
        return carry

    lax.fori_loop(0, n_chunks // 2, body, 0)


def kernel(tokens, table):
    n_b, n_j = tokens.shape
    idx = tokens.T.reshape(-1).astype(jnp.int32)
    b_total = idx.shape[0]
    b_per_w = b_total // _NW
    n_chunks = b_per_w // CHUNK
    n_bblk = n_b // CHUNK
    table8 = _prep_table(table)
    mesh = plsc.VectorSubcoreMesh(core_axis_name="c", subcore_axis_name="s")
    out5 = pl.kernel(
        functools.partial(_gather_body, b_per_w=b_per_w, n_chunks=n_chunks,
                          n_bblk=n_bblk),
        out_type=jax.ShapeDtypeStruct((n_j, 8, n_bblk, 8, CHUNK), jnp.float32),
        mesh=mesh,
        scratch_types=[
            pltpu.VMEM((b_per_w,), jnp.int32),
            pltpu.VMEM((CHUNK, D_PAD), jnp.float32),
            pltpu.VMEM((CHUNK, D_PAD), jnp.float32),
            pltpu.VMEM((8, 8, CHUNK), jnp.float32),
            pltpu.SemaphoreType.DMA,
            pltpu.SemaphoreType.DMA,
        ],
        compiler_params=pltpu.CompilerParams(
            use_tc_tiling_on_sc=False, needs_layout_passes=False
        ),
    )(idx, table8)
    return jnp.transpose(out5, (2, 4, 0, 1, 3)).reshape(n_b, n_j, D_MODEL)


# speedup vs baseline: 2.0208x; 2.0208x over previous
import functools

import jax
import jax.numpy as jnp
from jax import lax
from jax.experimental import pallas as pl
from jax.experimental.pallas import tpu as pltpu
from jax.experimental.pallas import tpu_sc as plsc

D_MODEL = 64
D_PAD = 128
SCALE = 8.0

_info = plsc.get_sparse_core_info()
_NC, _NS, _L = _info.num_cores, _info.num_subcores, _info.num_lanes
_NW = _NC * _NS

CHUNK = 128
BV = 4096


def _prep_body(tt_ref, out_ref):
    eye = (lax.broadcasted_iota(jnp.int32, (D_MODEL, D_MODEL), 0)
           == lax.broadcasted_iota(jnp.int32, (D_MODEL, D_MODEL), 1)
           ).astype(jnp.float32)
    at = lax.dot_general(tt_ref[...], eye * SCALE, (((0,), (0,)), ((), ())),
                         preferred_element_type=jnp.float32)
    out_ref[:, :D_MODEL] = at


def _prep_table(table):
    v = table.shape[0]
    tt = table.T
    return pl.pallas_call(
        _prep_body,
        grid=(pl.cdiv(v, BV),),
        in_specs=[pl.BlockSpec((D_MODEL, BV), lambda i: (0, i))],
        out_specs=pl.BlockSpec((BV, D_PAD), lambda i: (i, 0)),
        out_shape=jax.ShapeDtypeStruct((v, D_PAD), jnp.float32),
    )(tt)


def _gather_body(idx_hbm, table_hbm, out_hbm, idx_v, buf0, buf1, sem0, sem1,
                 *, b_per_w, n_chunks):
    wid = lax.axis_index("s") * _NC + lax.axis_index("c")
    base = wid * b_per_w
    pltpu.sync_copy(idx_hbm.at[pl.ds(base, b_per_w)], idx_v)

    bufs = (buf0, buf1)
    sems = (sem0, sem1)

    def start_gather(g, b):
        pltpu.make_async_copy(
            table_hbm.at[idx_v.at[pl.ds(g * CHUNK, CHUNK)]], bufs[b], sems[b]
        ).start()

    def finish(g, b):
        pltpu.make_async_copy(
            table_hbm.at[idx_v.at[pl.ds(g * CHUNK, CHUNK)]], bufs[b], sems[b]
        ).wait()
        pltpu.sync_copy(bufs[b], out_hbm.at[pl.ds(base + g * CHUNK, CHUNK)])
        @pl.when(g + 2 < n_chunks)
        def _():
            start_gather(g + 2, b)

    start_gather(0, 0)
    start_gather(1, 1)

    def body(p, carry):
        g = p * 2
        finish(g, 0)
        finish(g + 1, 1)
        return carry

    lax.fori_loop(0, n_chunks // 2, body, 0)


def kernel(tokens, table):
    idx = tokens.reshape(-1).astype(jnp.int32)
    b_total = idx.shape[0]
    b_per_w = b_total // _NW
    n_chunks = b_per_w // CHUNK
    table8 = _prep_table(table)
    mesh = plsc.VectorSubcoreMesh(core_axis_name="c", subcore_axis_name="s")
    out = pl.kernel(
        functools.partial(_gather_body, b_per_w=b_per_w, n_chunks=n_chunks),
        out_type=jax.ShapeDtypeStruct((b_total, D_PAD), jnp.float32),
        mesh=mesh,
        scratch_types=[
            pltpu.VMEM((b_per_w,), jnp.int32),
            pltpu.VMEM((CHUNK, D_PAD), jnp.float32),
            pltpu.VMEM((CHUNK, D_PAD), jnp.float32),
            pltpu.SemaphoreType.DMA,
            pltpu.SemaphoreType.DMA,
        ],
        compiler_params=pltpu.CompilerParams(use_tc_tiling_on_sc=True),
    )(idx, table8)
    return out[:, :D_MODEL].reshape(tokens.shape + (D_MODEL,))


# speedup vs baseline: 2.3759x; 1.1757x over previous
import functools

import jax
import jax.numpy as jnp
from jax import lax
from jax.experimental import pallas as pl
from jax.experimental.pallas import tpu as pltpu
from jax.experimental.pallas import tpu_sc as plsc

D_MODEL = 64
D_PAD = 128
SCALE = 8.0

_info = plsc.get_sparse_core_info()
_NC, _NS, _L = _info.num_cores, _info.num_subcores, _info.num_lanes
_NW = _NC * _NS

CHUNK = 128
BV = 8192


def _prep_body(tt_ref, out_ref):
    eye = (lax.broadcasted_iota(jnp.int32, (D_MODEL, D_MODEL), 0)
           == lax.broadcasted_iota(jnp.int32, (D_MODEL, D_MODEL), 1)
           ).astype(jnp.float32)
    at = lax.dot_general(tt_ref[...], eye * SCALE, (((0,), (0,)), ((), ())),
                         preferred_element_type=jnp.float32)
    out_ref[:, :D_MODEL] = at


def _prep_table(table):
    v = table.shape[0]
    tt = table.T
    return pl.pallas_call(
        _prep_body,
        grid=(pl.cdiv(v, BV),),
        in_specs=[pl.BlockSpec((D_MODEL, BV), lambda i: (0, i))],
        out_specs=pl.BlockSpec((BV, D_PAD), lambda i: (i, 0)),
        out_shape=jax.ShapeDtypeStruct((v, D_PAD), jnp.float32),
    )(tt)


def _gather_body(idx_hbm, table_hbm, out_hbm, idx_v, buf0, buf1, sem0, sem1,
                 *, b_per_w, n_chunks):
    wid = lax.axis_index("s") * _NC + lax.axis_index("c")
    base = wid * b_per_w
    pltpu.sync_copy(idx_hbm.at[pl.ds(base, b_per_w)], idx_v)

    bufs = (buf0, buf1)
    sems = (sem0, sem1)

    def start_gather(g, b):
        pltpu.make_async_copy(
            table_hbm.at[idx_v.at[pl.ds(g * CHUNK, CHUNK)]], bufs[b], sems[b]
        ).start()

    def finish(g, b):
        pltpu.make_async_copy(
            table_hbm.at[idx_v.at[pl.ds(g * CHUNK, CHUNK)]], bufs[b], sems[b]
        ).wait()
        pltpu.sync_copy(
            bufs[b].at[:, pl.ds(0, D_MODEL)],
            out_hbm.at[pl.ds(base + g * CHUNK, CHUNK), pl.ds(0, D_MODEL)],
        )
        @pl.when(g + 2 < n_chunks)
        def _():
            start_gather(g + 2, b)

    start_gather(0, 0)
    start_gather(1, 1)

    def body(p, carry):
        g = p * 2
        finish(g, 0)
        finish(g + 1, 1)
        return carry

    lax.fori_loop(0, n_chunks // 2, body, 0)


def kernel(tokens, table):
    idx = tokens.reshape(-1).astype(jnp.int32)
    b_total = idx.shape[0]
    b_per_w = b_total // _NW
    n_chunks = b_per_w // CHUNK
    table8 = _prep_table(table)
    mesh = plsc.VectorSubcoreMesh(core_axis_name="c", subcore_axis_name="s")
    out = pl.kernel(
        functools.partial(_gather_body, b_per_w=b_per_w, n_chunks=n_chunks),
        out_type=jax.ShapeDtypeStruct((b_total, D_PAD), jnp.float32),
        mesh=mesh,
        scratch_types=[
            pltpu.VMEM((b_per_w,), jnp.int32),
            pltpu.VMEM((CHUNK, D_PAD), jnp.float32),
            pltpu.VMEM((CHUNK, D_PAD), jnp.float32),
            pltpu.SemaphoreType.DMA,
            pltpu.SemaphoreType.DMA,
        ],
        compiler_params=pltpu.CompilerParams(
            use_tc_tiling_on_sc=False, needs_layout_passes=False
        ),
    )(idx, table8)
    return out[:, :D_MODEL].reshape(tokens.shape + (D_MODEL,))


# speedup vs baseline: 2.6460x; 1.1137x over previous
import functools

import jax
import jax.numpy as jnp
from jax import lax
from jax.experimental import pallas as pl
from jax.experimental.pallas import tpu as pltpu
from jax.experimental.pallas import tpu_sc as plsc

D_MODEL = 64
D_PAD = 128
SCALE = 8.0

_info = plsc.get_sparse_core_info()
_NC, _NS, _L = _info.num_cores, _info.num_subcores, _info.num_lanes
_NW = _NC * _NS

CHUNK = 128
BV = 8192


def _prep_body(tt_ref, out_ref):
    eye = (lax.broadcasted_iota(jnp.int32, (D_MODEL, D_MODEL), 0)
           == lax.broadcasted_iota(jnp.int32, (D_MODEL, D_MODEL), 1)
           ).astype(jnp.float32)
    at = lax.dot_general(tt_ref[...], eye * SCALE, (((0,), (0,)), ((), ())),
                         preferred_element_type=jnp.float32)
    out_ref[:, :D_MODEL] = at


def _prep_table(table):
    v = table.shape[0]
    tt = table.T
    return pl.pallas_call(
        _prep_body,
        grid=(pl.cdiv(v, BV),),
        in_specs=[pl.BlockSpec((D_MODEL, BV), lambda i: (0, i))],
        out_specs=pl.BlockSpec((BV, D_PAD), lambda i: (i, 0)),
        out_shape=jax.ShapeDtypeStruct((v, D_PAD), jnp.float32),
    )(tt)


def _gather_body(idx_hbm, table_hbm, out_hbm, idx_v, buf0, buf1, sem0, sem1,
                 *, b_per_w, n_chunks):
    wid = lax.axis_index("s") * _NC + lax.axis_index("c")
    base = wid * b_per_w
    pltpu.sync_copy(idx_hbm.at[pl.ds(base, b_per_w)], idx_v)

    bufs = (buf0, buf1)
    sems = (sem0, sem1)

    def start_gather(g, b):
        pltpu.make_async_copy(
            table_hbm.at[idx_v.at[pl.ds(g * CHUNK, CHUNK)]], bufs[b], sems[b]
        ).start()

    def finish(g, b):
        pltpu.make_async_copy(
            table_hbm.at[idx_v.at[pl.ds(g * CHUNK, CHUNK)]], bufs[b], sems[b]
        ).wait()
        pltpu.sync_copy(
            bufs[b],
            out_hbm.at[pl.ds(base + g * CHUNK, CHUNK), pl.ds(0, D_MODEL)],
        )
        @pl.when(g + 2 < n_chunks)
        def _():
            start_gather(g + 2, b)

    start_gather(0, 0)
    start_gather(1, 1)

    def body(p, carry):
        g = p * 2
        finish(g, 0)
        finish(g + 1, 1)
        return carry

    lax.fori_loop(0, n_chunks // 2, body, 0)


def kernel(tokens, table):
    idx = tokens.reshape(-1).astype(jnp.int32) * 2
    b_total = idx.shape[0]
    b_per_w = b_total // _NW
    n_chunks = b_per_w // CHUNK
    table8 = _prep_table(table).reshape(2 * table.shape[0], D_MODEL)
    mesh = plsc.VectorSubcoreMesh(core_axis_name="c", subcore_axis_name="s")
    out = pl.kernel(
        functools.partial(_gather_body, b_per_w=b_per_w, n_chunks=n_chunks),
        out_type=jax.ShapeDtypeStruct((b_total, D_PAD), jnp.float32),
        mesh=mesh,
        scratch_types=[
            pltpu.VMEM((b_per_w,), jnp.int32),
            pltpu.VMEM((CHUNK, D_MODEL), jnp.float32),
            pltpu.VMEM((CHUNK, D_MODEL), jnp.float32),
            pltpu.SemaphoreType.DMA,
            pltpu.SemaphoreType.DMA,
        ],
        compiler_params=pltpu.CompilerParams(
            use_tc_tiling_on_sc=False, needs_layout_passes=False
        ),
    )(idx, table8)
    return out[:, :D_MODEL].reshape(tokens.shape + (D_MODEL,))


# speedup vs baseline: 2.8340x; 1.0711x over previous
import functools

import jax
import jax.numpy as jnp
from jax import lax
from jax.experimental import pallas as pl
from jax.experimental.pallas import tpu as pltpu
from jax.experimental.pallas import tpu_sc as plsc

D_MODEL = 64
D_PAD = 128
SCALE = 8.0

_info = plsc.get_sparse_core_info()
_NC, _NS, _L = _info.num_cores, _info.num_subcores, _info.num_lanes
_NW = _NC * _NS

CHUNK = 128
BV = 8192


def _prep_body(tt1_ref, tt2_ref, out_ref):
    eye = (lax.broadcasted_iota(jnp.int32, (D_MODEL, D_MODEL), 0)
           == lax.broadcasted_iota(jnp.int32, (D_MODEL, D_MODEL), 1)
           ).astype(jnp.float32) * SCALE
    dims = (((0,), (0,)), ((), ()))
    out_ref[:, :D_MODEL] = lax.dot_general(
        tt1_ref[...], eye, dims, preferred_element_type=jnp.float32)
    out_ref[:, D_MODEL:] = lax.dot_general(
        tt2_ref[...], eye, dims, preferred_element_type=jnp.float32)


def _prep_table(table):
    v = table.shape[0]
    n_blk = pl.cdiv(v, 2 * BV)
    hi = pl.cdiv(v, BV) - 1
    tt = table.T
    return pl.pallas_call(
        _prep_body,
        grid=(n_blk,),
        in_specs=[
            pl.BlockSpec((D_MODEL, BV), lambda i: (0, jnp.minimum(2 * i, hi))),
            pl.BlockSpec(
                (D_MODEL, BV), lambda i: (0, jnp.minimum(2 * i + 1, hi))
            ),
        ],
        out_specs=pl.BlockSpec((BV, D_PAD), lambda i: (i, 0)),
        out_shape=jax.ShapeDtypeStruct((n_blk * BV, D_PAD), jnp.float32),
    )(tt, tt)


def _gather_body(idx_hbm, table_hbm, out_hbm, idx_v, buf0, buf1, sem0, sem1,
                 *, b_per_w, n_chunks):
    wid = lax.axis_index("s") * _NC + lax.axis_index("c")
    base = wid * b_per_w
    pltpu.sync_copy(idx_hbm.at[pl.ds(base, b_per_w)], idx_v)

    bufs = (buf0, buf1)
    sems = (sem0, sem1)

    def start_gather(g, b):
        pltpu.make_async_copy(
            table_hbm.at[idx_v.at[pl.ds(g * CHUNK, CHUNK)]], bufs[b], sems[b]
        ).start()

    def finish(g, b):
        pltpu.make_async_copy(
            table_hbm.at[idx_v.at[pl.ds(g * CHUNK, CHUNK)]], bufs[b], sems[b]
        ).wait()
        pltpu.sync_copy(
            bufs[b],
            out_hbm.at[pl.ds(base + g * CHUNK, CHUNK), pl.ds(0, D_MODEL)],
        )
        @pl.when(g + 2 < n_chunks)
        def _():
            start_gather(g + 2, b)

    start_gather(0, 0)
    start_gather(1, 1)

    def body(p, carry):
        g = p * 2
        finish(g, 0)
        finish(g + 1, 1)
        return carry

    lax.fori_loop(0, n_chunks // 2, body, 0)


def kernel(tokens, table):
    v = tokens.reshape(-1).astype(jnp.int32)
    idx = (v // (2 * BV)) * (2 * BV) + (v % BV) * 2 + ((v // BV) & 1)
    b_total = idx.shape[0]
    b_per_w = b_total // _NW
    n_chunks = b_per_w // CHUNK
    table8p = _prep_table(table)
    table8 = table8p.reshape(2 * table8p.shape[0], D_MODEL)
    mesh = plsc.VectorSubcoreMesh(core_axis_name="c", subcore_axis_name="s")
    out = pl.kernel(
        functools.partial(_gather_body, b_per_w=b_per_w, n_chunks=n_chunks),
        out_type=jax.ShapeDtypeStruct((b_total, D_PAD), jnp.float32),
        mesh=mesh,
        scratch_types=[
            pltpu.VMEM((b_per_w,), jnp.int32),
            pltpu.VMEM((CHUNK, D_MODEL), jnp.float32),
            pltpu.VMEM((CHUNK, D_MODEL), jnp.float32),
            pltpu.SemaphoreType.DMA,
            pltpu.SemaphoreType.DMA,
        ],
        compiler_params=pltpu.CompilerParams(
            use_tc_tiling_on_sc=False, needs_layout_passes=False
        ),
    )(idx, table8)
    return out[:, :D_MODEL].reshape(tokens.shape + (D_MODEL,))


# speedup vs baseline: 3.0179x; 1.0649x over previous
import functools

import jax
import jax.numpy as jnp
from jax import lax
from jax.experimental import pallas as pl
from jax.experimental.pallas import tpu as pltpu
from jax.experimental.pallas import tpu_sc as plsc

D_MODEL = 64
D_PAD = 128
SCALE = 8.0

_info = plsc.get_sparse_core_info()
_NC, _NS, _L = _info.num_cores, _info.num_subcores, _info.num_lanes
_NW = _NC * _NS

CHUNK = 256
BV = 16384


def _prep_body(tt1_ref, tt2_ref, out_ref):
    eye = (lax.broadcasted_iota(jnp.int32, (D_MODEL, D_MODEL), 0)
           == lax.broadcasted_iota(jnp.int32, (D_MODEL, D_MODEL), 1)
           ).astype(jnp.float32) * SCALE
    dims = (((0,), (0,)), ((), ()))
    out_ref[:, :D_MODEL] = lax.dot_general(
        tt1_ref[...], eye, dims, preferred_element_type=jnp.float32)
    out_ref[:, D_MODEL:] = lax.dot_general(
        tt2_ref[...], eye, dims, preferred_element_type=jnp.float32)


def _prep_table(table):
    v = table.shape[0]
    n_blk = pl.cdiv(v, 2 * BV)
    hi = pl.cdiv(v, BV) - 1
    tt = table.T
    return pl.pallas_call(
        _prep_body,
        grid=(n_blk,),
        in_specs=[
            pl.BlockSpec((D_MODEL, BV), lambda i: (0, jnp.minimum(2 * i, hi))),
            pl.BlockSpec(
                (D_MODEL, BV), lambda i: (0, jnp.minimum(2 * i + 1, hi))
            ),
        ],
        out_specs=pl.BlockSpec((BV, D_PAD), lambda i: (i, 0)),
        out_shape=jax.ShapeDtypeStruct((n_blk * BV, D_PAD), jnp.float32),
    )(tt, tt)


def _gather_body(idx_hbm, table_hbm, out_hbm, idx_v, buf0, buf1, sem0, sem1,
                 *, b_per_w, n_chunks):
    wid = lax.axis_index("s") * _NC + lax.axis_index("c")
    base = wid * b_per_w
    pltpu.sync_copy(idx_hbm.at[pl.ds(base, b_per_w)], idx_v)

    bufs = (buf0, buf1)
    sems = (sem0, sem1)

    def start_gather(g, b):
        pltpu.make_async_copy(
            table_hbm.at[idx_v.at[pl.ds(g * CHUNK, CHUNK)]], bufs[b], sems[b]
        ).start()

    def finish(g, b):
        pltpu.make_async_copy(
            table_hbm.at[idx_v.at[pl.ds(g * CHUNK, CHUNK)]], bufs[b], sems[b]
        ).wait()
        pltpu.sync_copy(
            bufs[b],
            out_hbm.at[pl.ds(base + g * CHUNK, CHUNK), pl.ds(0, D_MODEL)],
        )
        @pl.when(g + 2 < n_chunks)
        def _():
            start_gather(g + 2, b)

    start_gather(0, 0)
    start_gather(1, 1)

    def body(p, carry):
        g = p * 2
        finish(g, 0)
        finish(g + 1, 1)
        return carry

    lax.fori_loop(0, n_chunks // 2, body, 0)


def kernel(tokens, table):
    v = tokens.reshape(-1).astype(jnp.int32)
    idx = (v // (2 * BV)) * (2 * BV) + (v % BV) * 2 + ((v // BV) & 1)
    b_total = idx.shape[0]
    b_per_w = b_total // _NW
    n_chunks = b_per_w // CHUNK
    table8p = _prep_table(table)
    table8 = table8p.reshape(2 * table8p.shape[0], D_MODEL)
    mesh = plsc.VectorSubcoreMesh(core_axis_name="c", subcore_axis_name="s")
    out = pl.kernel(
        functools.partial(_gather_body, b_per_w=b_per_w, n_chunks=n_chunks),
        out_type=jax.ShapeDtypeStruct((b_total, D_PAD), jnp.float32),
        mesh=mesh,
        scratch_types=[
            pltpu.VMEM((b_per_w,), jnp.int32),
            pltpu.VMEM((CHUNK, D_MODEL), jnp.float32),
            pltpu.VMEM((CHUNK, D_MODEL), jnp.float32),
            pltpu.SemaphoreType.DMA,
            pltpu.SemaphoreType.DMA,
        ],
        compiler_params=pltpu.CompilerParams(
            use_tc_tiling_on_sc=False, needs_layout_passes=False
        ),
    )(idx, table8)
    return out[:, :D_MODEL].reshape(tokens.shape + (D_MODEL,))
